# baseline (speedup 1.0000x reference)
import jax
import jax.numpy as jnp
from jax.experimental import pallas as pl
from jax.experimental.pallas import tpu as pltpu

N = 4096
SIGMA = 0.2
EPSILON = 1.0
CUTOFF = 0.5
BR = 512
BC = 128
NBR = N // BR
NBC = N // BC
RPT = BR // BC
_SKIP2 = (CUTOFF + 1e-3) * (CUTOFF + 1e-3)


def _lj_row_kernel(pos_blk_ref, pos_all_ref, energy_ref, forces_ref,
                   acc_ref, bmin_ref, bmax_ref):
    ri = pl.program_id(0)

    @pl.when(ri == 0)
    def _():
        energy_ref[0, 0] = 0.0
        for c in range(NBC):
            sl = pos_all_ref[:, c * BC:(c + 1) * BC]
            for k in range(3):
                bmin_ref[c, k] = jnp.min(sl[k:k + 1, :])
                bmax_ref[c, k] = jnp.max(sl[k:k + 1, :])

    pos_blk = pos_blk_ref[...]
    px = pos_blk[:, 0:1]
    py = pos_blk[:, 1:2]
    pz = pos_blk[:, 2:3]

    acc_ref[...] = jnp.zeros_like(acc_ref)

    rminx = bmin_ref[ri * RPT, 0]
    rminy = bmin_ref[ri * RPT, 1]
    rminz = bmin_ref[ri * RPT, 2]
    rmaxx = bmax_ref[ri * RPT, 0]
    rmaxy = bmax_ref[ri * RPT, 1]
    rmaxz = bmax_ref[ri * RPT, 2]
    for t in range(1, RPT):
        rminx = jnp.minimum(rminx, bmin_ref[ri * RPT + t, 0])
        rminy = jnp.minimum(rminy, bmin_ref[ri * RPT + t, 1])
        rminz = jnp.minimum(rminz, bmin_ref[ri * RPT + t, 2])
        rmaxx = jnp.maximum(rmaxx, bmax_ref[ri * RPT + t, 0])
        rmaxy = jnp.maximum(rmaxy, bmax_ref[ri * RPT + t, 1])
        rmaxz = jnp.maximum(rmaxz, bmax_ref[ri * RPT + t, 2])

    for c in range(NBC):
        gx = jnp.maximum(jnp.maximum(bmin_ref[c, 0] - rmaxx,
                                     rminx - bmax_ref[c, 0]), 0.0)
        gy = jnp.maximum(jnp.maximum(bmin_ref[c, 1] - rmaxy,
                                     rminy - bmax_ref[c, 1]), 0.0)
        gz = jnp.maximum(jnp.maximum(bmin_ref[c, 2] - rmaxz,
                                     rminz - bmax_ref[c, 2]), 0.0)
        hit = gx * gx + gy * gy + gz * gz < _SKIP2

        @pl.when(hit)
        def _(c=c):
            pos_col = pos_all_ref[:, c * BC:(c + 1) * BC]
            dx = pos_col[0:1, :] - px
            dy = pos_col[1:2, :] - py
            dz = pos_col[2:3, :] - pz
            d2 = dx * dx + dy * dy + dz * dz
            r = jnp.sqrt(d2)

            valid = (r < CUTOFF) & (r > 0.0)

            idr = SIGMA / r
            idr2 = idr * idr
            idr6 = idr2 * idr2 * idr2
            idr12 = idr6 * idr6

            pair_e = jnp.where(valid, 4.0 * EPSILON * (idr12 - idr6), 0.0)
            pf = 24.0 * EPSILON / r * (2.0 * idr12 - idr6)
            g = jnp.where(valid, pf / r, 0.0)
            acc_ref[:, 0 * BC:1 * BC] += g * dx
            acc_ref[:, 1 * BC:2 * BC] += g * dy
            acc_ref[:, 2 * BC:3 * BC] += g * dz
            acc_ref[:, 3 * BC:4 * BC] += pair_e

    acc = acc_ref[...]
    fx = 2.0 * jnp.sum(acc[:, 0 * BC:1 * BC], axis=1, keepdims=True)
    fy = 2.0 * jnp.sum(acc[:, 1 * BC:2 * BC], axis=1, keepdims=True)
    fz = 2.0 * jnp.sum(acc[:, 2 * BC:3 * BC], axis=1, keepdims=True)
    forces_ref[...] = jnp.concatenate([fx, fy, fz], axis=1)
    energy_ref[0, 0] += 0.5 * jnp.sum(acc[:, 3 * BC:4 * BC])


@jax.jit
def kernel(positions, cell):
    del cell

    iota = jnp.arange(N, dtype=jnp.int32)
    sx, sy, sz, sperm = jax.lax.sort(
        (positions[:, 0], positions[:, 1], positions[:, 2], iota), num_keys=1)
    pos_s = jnp.stack([sx, sy, sz], axis=1)
    pos_t = jnp.stack([sx, sy, sz], axis=0)

    energy, forces_s = pl.pallas_call(
        _lj_row_kernel,
        grid=(NBR,),
        in_specs=[
            pl.BlockSpec((BR, 3), lambda r: (r, 0)),
            pl.BlockSpec((3, N), lambda r: (0, 0)),
        ],
        out_specs=[
            pl.BlockSpec(memory_space=pltpu.SMEM, block_shape=(1, 1),
                         index_map=lambda r: (0, 0)),
            pl.BlockSpec((BR, 3), lambda r: (r, 0)),
        ],
        out_shape=[
            jax.ShapeDtypeStruct((1, 1), jnp.float32),
            jax.ShapeDtypeStruct((N, 3), jnp.float32),
        ],
        scratch_shapes=[pltpu.VMEM((BR, 4 * BC), jnp.float32),
                        pltpu.SMEM((NBC, 3), jnp.float32),
                        pltpu.SMEM((NBC, 3), jnp.float32)],
    )(pos_s, pos_t)

    _, fx0, fy0, fz0 = jax.lax.sort(
        (sperm, forces_s[:, 0], forces_s[:, 1], forces_s[:, 2]), num_keys=1)
    forces = jnp.stack([fx0, fy0, fz0], axis=1)
    return energy[0, 0], forces


# speedup vs baseline: 1.4152x; 1.4152x over previous
import jax
import jax.numpy as jnp
from jax.experimental import pallas as pl
from jax.experimental.pallas import tpu as pltpu

N = 4096
SIGMA = 0.2
EPSILON = 1.0
CUTOFF = 0.5
BR = 256
BC = 256
NBR = N // BR
NBC = N // BC
_SKIP2 = (CUTOFF + 1e-3) * (CUTOFF + 1e-3)


def _lj_row_kernel(pos_blk_ref, pos_all_ref, energy_ref, forces_ref,
                   fcol_ref, acc_ref, bmin_ref, bmax_ref):
    ri = pl.program_id(0)

    @pl.when(ri == 0)
    def _():
        energy_ref[0, 0] = 0.0
        fcol_ref[...] = jnp.zeros_like(fcol_ref)
        for c in range(NBC):
            sl = pos_all_ref[:, c * BC:(c + 1) * BC]
            for k in range(3):
                bmin_ref[c, k] = jnp.min(sl[k:k + 1, :])
                bmax_ref[c, k] = jnp.max(sl[k:k + 1, :])

    pos_blk = pos_blk_ref[...]
    px = pos_blk[:, 0:1]
    py = pos_blk[:, 1:2]
    pz = pos_blk[:, 2:3]

    acc_ref[...] = jnp.zeros_like(acc_ref)

    for c in range(NBC):
        gx = jnp.maximum(jnp.maximum(bmin_ref[c, 0] - bmax_ref[ri, 0],
                                     bmin_ref[ri, 0] - bmax_ref[c, 0]), 0.0)
        gy = jnp.maximum(jnp.maximum(bmin_ref[c, 1] - bmax_ref[ri, 1],
                                     bmin_ref[ri, 1] - bmax_ref[c, 1]), 0.0)
        gz = jnp.maximum(jnp.maximum(bmin_ref[c, 2] - bmax_ref[ri, 2],
                                     bmin_ref[ri, 2] - bmax_ref[c, 2]), 0.0)
        hit = gx * gx + gy * gy + gz * gz < _SKIP2
        on_diag = c == ri

        @pl.when(jnp.logical_or(on_diag, jnp.logical_and(c > ri, hit)))
        def _(c=c, on_diag=on_diag):
            pos_col = pos_all_ref[:, c * BC:(c + 1) * BC]
            dx = pos_col[0:1, :] - px
            dy = pos_col[1:2, :] - py
            dz = pos_col[2:3, :] - pz
            d2 = dx * dx + dy * dy + dz * dz
            r = jnp.sqrt(d2)

            valid = (r < CUTOFF) & (r > 0.0)

            idr = SIGMA / r
            idr2 = idr * idr
            idr6 = idr2 * idr2 * idr2
            idr12 = idr6 * idr6

            pair_e = jnp.where(valid, 4.0 * EPSILON * (idr12 - idr6), 0.0)
            pf = 24.0 * EPSILON / r * (2.0 * idr12 - idr6)
            g = jnp.where(valid, pf / r, 0.0)
            gdx = g * dx
            gdy = g * dy
            gdz = g * dz
            ew = jnp.where(on_diag, 0.5, 1.0)
            acc_ref[:, 0 * BC:1 * BC] += gdx
            acc_ref[:, 1 * BC:2 * BC] += gdy
            acc_ref[:, 2 * BC:3 * BC] += gdz
            acc_ref[:, 3 * BC:4 * BC] += ew * pair_e

            @pl.when(c > ri)
            def _():
                fcol_ref[0:1, c * BC:(c + 1) * BC] += (-2.0) * jnp.sum(
                    gdx, axis=0, keepdims=True)
                fcol_ref[1:2, c * BC:(c + 1) * BC] += (-2.0) * jnp.sum(
                    gdy, axis=0, keepdims=True)
                fcol_ref[2:3, c * BC:(c + 1) * BC] += (-2.0) * jnp.sum(
                    gdz, axis=0, keepdims=True)

    acc = acc_ref[...]
    fx = 2.0 * jnp.sum(acc[:, 0 * BC:1 * BC], axis=1, keepdims=True)
    fy = 2.0 * jnp.sum(acc[:, 1 * BC:2 * BC], axis=1, keepdims=True)
    fz = 2.0 * jnp.sum(acc[:, 2 * BC:3 * BC], axis=1, keepdims=True)
    forces_ref[...] = jnp.concatenate([fx, fy, fz], axis=1)
    energy_ref[0, 0] += jnp.sum(acc[:, 3 * BC:4 * BC])


@jax.jit
def kernel(positions, cell):
    del cell

    iota = jnp.arange(N, dtype=jnp.int32)
    sx, sy, sz, sperm = jax.lax.sort(
        (positions[:, 0], positions[:, 1], positions[:, 2], iota), num_keys=1)
    pos_s = jnp.stack([sx, sy, sz], axis=1)
    pos_t = jnp.stack([sx, sy, sz], axis=0)

    energy, forces_s, fcol = pl.pallas_call(
        _lj_row_kernel,
        grid=(NBR,),
        in_specs=[
            pl.BlockSpec((BR, 3), lambda r: (r, 0)),
            pl.BlockSpec((3, N), lambda r: (0, 0)),
        ],
        out_specs=[
            pl.BlockSpec(memory_space=pltpu.SMEM, block_shape=(1, 1),
                         index_map=lambda r: (0, 0)),
            pl.BlockSpec((BR, 3), lambda r: (r, 0)),
            pl.BlockSpec((4, N), lambda r: (0, 0)),
        ],
        out_shape=[
            jax.ShapeDtypeStruct((1, 1), jnp.float32),
            jax.ShapeDtypeStruct((N, 3), jnp.float32),
            jax.ShapeDtypeStruct((4, N), jnp.float32),
        ],
        scratch_shapes=[pltpu.VMEM((BR, 4 * BC), jnp.float32),
                        pltpu.SMEM((NBC, 3), jnp.float32),
                        pltpu.SMEM((NBC, 3), jnp.float32)],
    )(pos_s, pos_t)

    _, fx0, fy0, fz0 = jax.lax.sort(
        (sperm,
         forces_s[:, 0] + fcol[0, :],
         forces_s[:, 1] + fcol[1, :],
         forces_s[:, 2] + fcol[2, :]), num_keys=1)
    forces = jnp.stack([fx0, fy0, fz0], axis=1)
    return energy[0, 0], forces
